# Initial kernel scaffold; baseline (speedup 1.0000x reference)
import jax
import jax.numpy as jnp
from jax.experimental import pallas as pl


def kernel(h, p, e, snorm_n, edge_index, graph_ids, params):
    raise NotImplementedError("write your pallas kernel here")


# speedup vs baseline: 1.0002x; 1.0002x over previous
import jax
import jax.numpy as jnp
from jax.experimental import pallas as pl

N = 50000
G = 128
HID = 64
PE = 16


def _apply_lin(lp, x):
    return x @ lp['w'] + lp['b']


def _mlp_head_body(hg_ref, w0_ref, b0_ref, w1_ref, b1_ref, w2_ref, b2_ref, o_ref):
    y = jnp.maximum(hg_ref[...] @ w0_ref[...] + b0_ref[...], 0.0)
    y = jnp.maximum(y @ w1_ref[...] + b1_ref[...], 0.0)
    o_ref[...] = y @ w2_ref[...] + b2_ref[...]


def _mlp_head(hg, params):
    return pl.pallas_call(
        _mlp_head_body,
        out_shape=jax.ShapeDtypeStruct((G, 1), jnp.float32),
    )(hg,
      params['mlp0']['w'], params['mlp0']['b'][None, :],
      params['mlp1']['w'], params['mlp1']['b'][None, :],
      params['mlp2']['w'], params['mlp2']['b'][None, :])


def _layer(lp, src, dst, h, p, e):
    h_in, p_in, e_in = h, p, e
    hp = jnp.concatenate([h, p], axis=-1)
    A1h = _apply_lin(lp['A1'], hp)
    A2hp = _apply_lin(lp['A2'], hp)
    B1h = _apply_lin(lp['B1'], hp)
    B2h = _apply_lin(lp['B2'], hp)
    B3e = _apply_lin(lp['B3'], e)
    C1p = _apply_lin(lp['C1'], p)
    C2p = _apply_lin(lp['C2'], p)
    hat_eta = B1h[src] + B2h[dst] + B3e
    sigma = jax.nn.sigmoid(hat_eta)
    sum_sigma = jax.ops.segment_sum(sigma, dst, num_segments=N)
    eta = sigma / (sum_sigma[dst] + 1e-6)
    h_new = A1h + jax.ops.segment_sum(eta * A2hp[src], dst, num_segments=N)
    p_new = C1p + jax.ops.segment_sum(eta * C2p[src], dst, num_segments=N)
    h = h_in + jax.nn.relu(h_new)
    p = p_in + jnp.tanh(p_new)
    e = e_in + jax.nn.relu(hat_eta)
    return h, p, e


def kernel(h, p, e, snorm_n, edge_index, graph_ids, params):
    src, dst = edge_index[0], edge_index[1]
    h = params['emb_h'][h]
    p = _apply_lin(params['emb_p'], p)
    e = params['emb_e'][e]
    for lp in params['layers']:
        h, p, e = _layer(lp, src, dst, h, p, e)
    p = _apply_lin(params['p_out'], p)
    counts = jnp.maximum(jnp.bincount(graph_ids, length=G), 1).astype(jnp.float32)
    means = jax.ops.segment_sum(p, graph_ids, num_segments=G) / counts[:, None]
    p = p - means[graph_ids]
    norms = jnp.sqrt(jax.ops.segment_sum(p * p, graph_ids, num_segments=G))
    p = p / norms[graph_ids]
    hp = _apply_lin(params['Whp'], jnp.concatenate([h, p], axis=-1))
    hg = jax.ops.segment_sum(hp, graph_ids, num_segments=G) / counts[:, None]
    return _mlp_head(hg, params)


# speedup vs baseline: 2.0489x; 2.0485x over previous
import functools

import jax
import jax.numpy as jnp
from jax import lax
from jax.experimental import pallas as pl
from jax.experimental.pallas import tpu as pltpu
from jax.experimental.pallas import tpu_sc as plsc

N = 50000
E = 800000
G = 128
HID = 64
PE = 16
QW = 16

NP = 50176
EP = 802816
TECS = 16
EP_TEC = EP // TECS
CH = 512
NSUB = CH // 128
NCH = EP_TEC // CH
STR = NP // TECS
ZR = 784

NB = 1792
GRID_N = NP // NB
EB = 3584
GRID_E = EP // EB

_MESH = dict(core_axis_name="c", subcore_axis_name="s")
F32 = jnp.float32
_SCP = pltpu.CompilerParams(use_tc_tiling_on_sc=False)


def _dotr(a, b):
    return lax.dot_general(a, b, (((0,), (0,)), ((), ())),
                           precision=lax.Precision.HIGHEST,
                           preferred_element_type=F32)


def _dot(a, b):
    return jnp.dot(a, b, precision=lax.Precision.HIGHEST,
                   preferred_element_type=F32)


def _dotx(a, b):
    return jnp.dot(a, b, precision=lax.Precision.HIGHEST,
                   preferred_element_type=F32)


def _zero_stripe(zb, sh, s):
    def zr(r, _):
        zb[r, pl.ds(0, 16)] = jnp.zeros((16,), F32)
        return 0
    lax.fori_loop(0, ZR, zr, 0)
    for q in range(4):
        pltpu.sync_copy(zb, sh.at[pl.ds(s * STR + q * ZR, ZR)])


def _dump_stripe(sh, acc_h, s, qN):
    for q in range(4):
        pltpu.sync_copy(sh.at[pl.ds(s * STR + q * ZR, ZR)],
                        acc_h.at[pl.ds(qN + s * STR + q * ZR, ZR)])


def _load_adjust_idx(src2d, dst2d, sidx, didx, asrc, adst, r0, qN, need_src):
    pltpu.sync_copy(dst2d.at[pl.ds(r0, NSUB)], didx)
    if need_src:
        pltpu.sync_copy(src2d.at[pl.ds(r0, NSUB)], sidx)
    for j in range(NSUB):
        for t in range(8):
            sl = pl.ds(t * 16, 16)
            if need_src:
                asrc[j, sl] = sidx[j, sl] + qN
            adst[j, sl] = didx[j, sl] + qN


def _sc_pass1(src2d, dst2d, b1f, b2f, b3f):
    @functools.partial(
        pl.kernel,
        out_type=(jax.ShapeDtypeStruct((EP, HID), F32),
                  jax.ShapeDtypeStruct((4 * NP, QW), F32)),
        mesh=plsc.VectorSubcoreMesh(**_MESH),
        compiler_params=_SCP,
        scratch_types=[
            pltpu.VMEM((NSUB, 128), jnp.int32),
            pltpu.VMEM((NSUB, 128), jnp.int32),
            pltpu.VMEM((NSUB, 128), jnp.int32),
            pltpu.VMEM((NSUB, 128), jnp.int32),
            pltpu.VMEM((CH, QW), F32),
            pltpu.VMEM((CH, QW), F32),
            pltpu.VMEM((CH, QW), F32),
            pltpu.VMEM((CH, QW), F32),
            pltpu.VMEM((ZR, QW), F32),
            pltpu.VMEM_SHARED((NP, QW), F32),
            pltpu.SemaphoreType.DMA,
            pltpu.SemaphoreType.DMA,
            pltpu.SemaphoreType.DMA,
        ],
    )
    def k(src_h, dst_h, b1_h, b2_h, b3_h, hat_h, ss_h,
          sidx, didx, asrc, adst, bA, bB, bC, bS, zb, sh, m1, m2, m3):
        c = lax.axis_index("c")
        s = lax.axis_index("s")
        for sub in range(2):
            qid = c * 2 + sub
            qN = qid * NP
            _zero_stripe(zb, sh, s)
            plsc.subcore_barrier()

            def chunk(kk, _):
                base = s * EP_TEC + kk * CH
                r0 = s * (EP_TEC // 128) + kk * NSUB
                _load_adjust_idx(src2d=src_h, dst2d=dst_h, sidx=sidx,
                                 didx=didx, asrc=asrc, adst=adst,
                                 r0=r0, qN=qN, need_src=True)
                cps = []
                for j in range(NSUB):
                    cps.append(pltpu.async_copy(
                        b1_h.at[asrc.at[j]],
                        bA.at[pl.ds(j * 128, 128)], m1))
                    cps.append(pltpu.async_copy(
                        b2_h.at[adst.at[j]],
                        bB.at[pl.ds(j * 128, 128)], m2))
                cps.append(pltpu.async_copy(
                    b3_h.at[pl.ds(base, CH), pl.ds(qid * QW, QW)], bC, m3))
                for cp in cps:
                    cp.wait()

                def row(r, _):
                    v = bA[r, pl.ds(0, 16)] + bB[r, pl.ds(0, 16)] \
                        + bC[r, pl.ds(0, 16)]
                    bC[r, pl.ds(0, 16)] = v
                    bS[r, pl.ds(0, 16)] = 1.0 / (1.0 + jnp.exp(-v))
                    return 0
                lax.fori_loop(0, CH, row, 0)
                pltpu.sync_copy(bC,
                                hat_h.at[pl.ds(base, CH), pl.ds(qid * QW, QW)])
                for j in range(NSUB):
                    pltpu.sync_copy(bS.at[pl.ds(j * 128, 128)],
                                    sh.at[didx.at[j]], add=True)
                return 0
            lax.fori_loop(0, NCH, chunk, 0)
            plsc.subcore_barrier()
            _dump_stripe(sh, ss_h, s, qN)
            plsc.subcore_barrier()

    return k(src2d, dst2d, b1f, b2f, b3f)


def _sc_pass2(src2d, dst2d, hatf, ssf, a2f):
    @functools.partial(
        pl.kernel,
        out_type=(jax.ShapeDtypeStruct((EP, HID), F32),
                  jax.ShapeDtypeStruct((4 * NP, QW), F32)),
        mesh=plsc.VectorSubcoreMesh(**_MESH),
        compiler_params=_SCP,
        scratch_types=[
            pltpu.VMEM((NSUB, 128), jnp.int32),
            pltpu.VMEM((NSUB, 128), jnp.int32),
            pltpu.VMEM((NSUB, 128), jnp.int32),
            pltpu.VMEM((NSUB, 128), jnp.int32),
            pltpu.VMEM((CH, QW), F32),
            pltpu.VMEM((CH, QW), F32),
            pltpu.VMEM((CH, QW), F32),
            pltpu.VMEM((CH, QW), F32),
            pltpu.VMEM((ZR, QW), F32),
            pltpu.VMEM_SHARED((NP, QW), F32),
            pltpu.SemaphoreType.DMA,
            pltpu.SemaphoreType.DMA,
            pltpu.SemaphoreType.DMA,
        ],
    )
    def k(src_h, dst_h, hat_h, ssn_h, a2_h, eta_h, acc_h,
          sidx, didx, asrc, adst, bA, bB, bC, bS, zb, sh, m1, m2, m3):
        c = lax.axis_index("c")
        s = lax.axis_index("s")
        for sub in range(2):
            qid = c * 2 + sub
            qN = qid * NP
            _zero_stripe(zb, sh, s)
            plsc.subcore_barrier()

            def chunk(kk, _):
                base = s * EP_TEC + kk * CH
                r0 = s * (EP_TEC // 128) + kk * NSUB
                _load_adjust_idx(src2d=src_h, dst2d=dst_h, sidx=sidx,
                                 didx=didx, asrc=asrc, adst=adst,
                                 r0=r0, qN=qN, need_src=True)
                cps = []
                for j in range(NSUB):
                    cps.append(pltpu.async_copy(
                        a2_h.at[asrc.at[j]],
                        bA.at[pl.ds(j * 128, 128)], m1))
                    cps.append(pltpu.async_copy(
                        ssn_h.at[adst.at[j]],
                        bB.at[pl.ds(j * 128, 128)], m2))
                cps.append(pltpu.async_copy(
                    hat_h.at[pl.ds(base, CH), pl.ds(qid * QW, QW)], bC, m3))
                for cp in cps:
                    cp.wait()

                def row(r, _):
                    v = bC[r, pl.ds(0, 16)]
                    sg = 1.0 / (1.0 + jnp.exp(-v))
                    eta = sg / (bB[r, pl.ds(0, 16)] + 1e-6)
                    bC[r, pl.ds(0, 16)] = eta
                    bS[r, pl.ds(0, 16)] = eta * bA[r, pl.ds(0, 16)]
                    return 0
                lax.fori_loop(0, CH, row, 0)
                pltpu.sync_copy(bC,
                                eta_h.at[pl.ds(base, CH), pl.ds(qid * QW, QW)])
                for j in range(NSUB):
                    pltpu.sync_copy(bS.at[pl.ds(j * 128, 128)],
                                    sh.at[didx.at[j]], add=True)
                return 0
            lax.fori_loop(0, NCH, chunk, 0)
            plsc.subcore_barrier()
            _dump_stripe(sh, acc_h, s, qN)
            plsc.subcore_barrier()

    return k(src2d, dst2d, hatf, ssf, a2f)


def _sc_pass3(src2d, dst2d, etaf, c2f):
    @functools.partial(
        pl.kernel,
        out_type=jax.ShapeDtypeStruct((4 * NP, QW), F32),
        mesh=plsc.VectorSubcoreMesh(**_MESH),
        compiler_params=_SCP,
        scratch_types=[
            pltpu.VMEM((NSUB, 128), jnp.int32),
            pltpu.VMEM((NSUB, 128), jnp.int32),
            pltpu.VMEM((NSUB, 128), jnp.int32),
            pltpu.VMEM((NSUB, 128), jnp.int32),
            pltpu.VMEM((CH, QW), F32),
            pltpu.VMEM((CH, QW), F32),
            pltpu.VMEM((CH, QW), F32),
            pltpu.VMEM((ZR, QW), F32),
            pltpu.VMEM_SHARED((NP, QW), F32),
            pltpu.SemaphoreType.DMA,
            pltpu.SemaphoreType.DMA,
        ],
    )
    def k(src_h, dst_h, eta_h, c2_h, acc_h,
          sidx, didx, asrc, adst, bA, bC, bS, zb, sh, m1, m3):
        c = lax.axis_index("c")
        s = lax.axis_index("s")
        for sub in range(2):
            qid = c * 2 + sub
            qN = qid * NP
            _zero_stripe(zb, sh, s)
            plsc.subcore_barrier()

            def chunk(kk, _):
                base = s * EP_TEC + kk * CH
                r0 = s * (EP_TEC // 128) + kk * NSUB
                _load_adjust_idx(src2d=src_h, dst2d=dst_h, sidx=sidx,
                                 didx=didx, asrc=asrc, adst=adst,
                                 r0=r0, qN=qN, need_src=True)
                cps = []
                for j in range(NSUB):
                    cps.append(pltpu.async_copy(
                        c2_h.at[asrc.at[j]],
                        bA.at[pl.ds(j * 128, 128)], m1))
                cps.append(pltpu.async_copy(
                    eta_h.at[pl.ds(base, CH), pl.ds(qid * QW, QW)], bC, m3))
                for cp in cps:
                    cp.wait()

                def row(r, _):
                    bS[r, pl.ds(0, 16)] = bC[r, pl.ds(0, 16)] \
                        * bA[r, pl.ds(0, 16)]
                    return 0
                lax.fori_loop(0, CH, row, 0)
                for j in range(NSUB):
                    pltpu.sync_copy(bS.at[pl.ds(j * 128, 128)],
                                    sh.at[didx.at[j]], add=True)
                return 0
            lax.fori_loop(0, NCH, chunk, 0)
            plsc.subcore_barrier()
            _dump_stripe(sh, acc_h, s, qN)
            plsc.subcore_barrier()

    return k(src2d, dst2d, etaf, c2f)


def _split4(ref, t):
    for q in range(4):
        ref[q, :, :] = t[:, q * QW:(q + 1) * QW]


def _k_embed(ids2d, p_pad, emb_h, wp, bp):
    def body(ids_ref, p_ref, eh_ref, wp_ref, bp_ref, h0_ref, p0_ref):
        oh = (ids_ref[...] ==
              lax.broadcasted_iota(jnp.int32, (1, 28), 1)).astype(F32)
        h0_ref[...] = _dotx(oh, eh_ref[...])
        p0_ref[...] = _dot(p_ref[...], wp_ref[...]) + bp_ref[...]

    return pl.pallas_call(
        body,
        grid=(GRID_N,),
        in_specs=[
            pl.BlockSpec((NB, 1), lambda i: (i, 0)),
            pl.BlockSpec((NB, PE), lambda i: (i, 0)),
            pl.BlockSpec((28, HID), lambda i: (0, 0)),
            pl.BlockSpec((PE, HID), lambda i: (0, 0)),
            pl.BlockSpec((1, HID), lambda i: (0, 0)),
        ],
        out_specs=[
            pl.BlockSpec((NB, HID), lambda i: (i, 0)),
            pl.BlockSpec((NB, HID), lambda i: (i, 0)),
        ],
        out_shape=[
            jax.ShapeDtypeStruct((NP, HID), F32),
            jax.ShapeDtypeStruct((NP, HID), F32),
        ],
    )(ids2d, p_pad, emb_h, wp, bp)


def _k_node_pre(h, p, lp):
    def body(h_ref, p_ref, wa1, ba1, wa2, ba2, wb1, bb1, wb2, bb2,
             wc1, bc1, wc2, bc2, a1_ref, c1_ref, b1s, b2s, a2s, c2s):
        hp = jnp.concatenate([h_ref[...], p_ref[...]], axis=1)
        a1_ref[...] = _dot(hp, wa1[...]) + ba1[...]
        c1_ref[...] = _dot(p_ref[...], wc1[...]) + bc1[...]
        _split4(b1s, _dot(hp, wb1[...]) + bb1[...])
        _split4(b2s, _dot(hp, wb2[...]) + bb2[...])
        _split4(a2s, _dot(hp, wa2[...]) + ba2[...])
        _split4(c2s, _dot(p_ref[...], wc2[...]) + bc2[...])

    wspec = lambda shp: pl.BlockSpec(shp, lambda i: (0, 0))
    nspec = pl.BlockSpec((NB, HID), lambda i: (i, 0))
    sspec = pl.BlockSpec((4, NB, QW), lambda i: (0, i, 0))
    return pl.pallas_call(
        body,
        grid=(GRID_N,),
        in_specs=[nspec, nspec,
                  wspec((2 * HID, HID)), wspec((1, HID)),
                  wspec((2 * HID, HID)), wspec((1, HID)),
                  wspec((2 * HID, HID)), wspec((1, HID)),
                  wspec((2 * HID, HID)), wspec((1, HID)),
                  wspec((HID, HID)), wspec((1, HID)),
                  wspec((HID, HID)), wspec((1, HID))],
        out_specs=[nspec, nspec, sspec, sspec, sspec, sspec],
        out_shape=[
            jax.ShapeDtypeStruct((NP, HID), F32),
            jax.ShapeDtypeStruct((NP, HID), F32),
            jax.ShapeDtypeStruct((4, NP, QW), F32),
            jax.ShapeDtypeStruct((4, NP, QW), F32),
            jax.ShapeDtypeStruct((4, NP, QW), F32),
            jax.ShapeDtypeStruct((4, NP, QW), F32),
        ],
    )(h, p,
      lp['A1']['w'], lp['A1']['b'][None, :],
      lp['A2']['w'], lp['A2']['b'][None, :],
      lp['B1']['w'], lp['B1']['b'][None, :],
      lp['B2']['w'], lp['B2']['b'][None, :],
      lp['C1']['w'], lp['C1']['b'][None, :],
      lp['C2']['w'], lp['C2']['b'][None, :])


def _k_edge_embed(eids2d, emb_e, w3, b3):
    def body(ids_ref, ee_ref, w3_ref, b3_ref, es_ref, b3s_ref):
        oh = (ids_ref[...] ==
              lax.broadcasted_iota(jnp.int32, (1, 4), 1)).astype(F32)
        e1 = _dotx(oh, ee_ref[...])
        es_ref[...] = e1
        b3s_ref[...] = _dot(e1, w3_ref[...]) + b3_ref[...]

    espec = pl.BlockSpec((EB, HID), lambda i: (i, 0))
    return pl.pallas_call(
        body,
        grid=(GRID_E,),
        in_specs=[
            pl.BlockSpec((EB, 1), lambda i: (i, 0)),
            pl.BlockSpec((4, HID), lambda i: (0, 0)),
            pl.BlockSpec((HID, HID), lambda i: (0, 0)),
            pl.BlockSpec((1, HID), lambda i: (0, 0)),
        ],
        out_specs=[espec, espec],
        out_shape=[
            jax.ShapeDtypeStruct((EP, HID), F32),
            jax.ShapeDtypeStruct((EP, HID), F32),
        ],
    )(eids2d, emb_e, w3, b3)


def _k_edge_update(es, hats, w3, b3):
    def body(es_ref, ht_ref, w3_ref, b3_ref, es2_ref, b3s_ref):
        e2 = es_ref[...] + jnp.maximum(ht_ref[...], 0.0)
        es2_ref[...] = e2
        b3s_ref[...] = _dot(e2, w3_ref[...]) + b3_ref[...]

    espec = pl.BlockSpec((EB, HID), lambda i: (i, 0))
    return pl.pallas_call(
        body,
        grid=(GRID_E,),
        in_specs=[espec, espec,
                  pl.BlockSpec((HID, HID), lambda i: (0, 0)),
                  pl.BlockSpec((1, HID), lambda i: (0, 0))],
        out_specs=[espec, espec],
        out_shape=[
            jax.ShapeDtypeStruct((EP, HID), F32),
            jax.ShapeDtypeStruct((EP, HID), F32),
        ],
    )(es, hats, w3, b3)


def _k_node_update(h, p, a1h, c1p, acchs, accps):
    def body(h_ref, p_ref, a1_ref, c1_ref, ah_ref, ap_ref, h2_ref, p2_ref):
        acch = jnp.concatenate([ah_ref[q, :, :] for q in range(4)], axis=1)
        accp = jnp.concatenate([ap_ref[q, :, :] for q in range(4)], axis=1)
        h2_ref[...] = h_ref[...] + jnp.maximum(a1_ref[...] + acch, 0.0)
        p2_ref[...] = p_ref[...] + jnp.tanh(c1_ref[...] + accp)

    nspec = pl.BlockSpec((NB, HID), lambda i: (i, 0))
    sspec = pl.BlockSpec((4, NB, QW), lambda i: (0, i, 0))
    return pl.pallas_call(
        body,
        grid=(GRID_N,),
        in_specs=[nspec, nspec, nspec, nspec, sspec, sspec],
        out_specs=[nspec, nspec],
        out_shape=[
            jax.ShapeDtypeStruct((NP, HID), F32),
            jax.ShapeDtypeStruct((NP, HID), F32),
        ],
    )(h, p, a1h, c1p, acchs, accps)


def _oh_g(gid_ref):
    return (gid_ref[...] ==
            lax.broadcasted_iota(jnp.int32, (1, G), 1)).astype(F32)


def _k_read1(p4, gid2d, wpo, bpo):
    def body(p4_ref, gid_ref, w_ref, b_ref, pl_ref, sums_ref, cnt_ref):
        i = pl.program_id(0)
        oh = _oh_g(gid_ref)
        plv = _dot(p4_ref[...], w_ref[...]) + b_ref[...]
        pl_ref[...] = plv

        @pl.when(i == 0)
        def _():
            sums_ref[...] = jnp.zeros_like(sums_ref)
            cnt_ref[...] = jnp.zeros_like(cnt_ref)
        sums_ref[...] += _dotr(oh, plv)
        cnt_ref[...] += _dotr(oh, jnp.ones((NB, 8), F32))

    return pl.pallas_call(
        body,
        grid=(GRID_N,),
        in_specs=[
            pl.BlockSpec((NB, HID), lambda i: (i, 0)),
            pl.BlockSpec((NB, 1), lambda i: (i, 0)),
            pl.BlockSpec((HID, PE), lambda i: (0, 0)),
            pl.BlockSpec((1, PE), lambda i: (0, 0)),
        ],
        out_specs=[
            pl.BlockSpec((NB, PE), lambda i: (i, 0)),
            pl.BlockSpec((G, PE), lambda i: (0, 0)),
            pl.BlockSpec((G, 8), lambda i: (0, 0)),
        ],
        out_shape=[
            jax.ShapeDtypeStruct((NP, PE), F32),
            jax.ShapeDtypeStruct((G, PE), F32),
            jax.ShapeDtypeStruct((G, 8), F32),
        ],
    )(p4, gid2d, wpo, bpo)


def _k_read2(pL, gid2d, sums, cnt):
    def body(pl_ref, gid_ref, sums_ref, cnt_ref, pc_ref, nsq_ref):
        i = pl.program_id(0)
        oh = _oh_g(gid_ref)
        means = sums_ref[...] / jnp.maximum(cnt_ref[:, :1], 1.0)
        pc = pl_ref[...] - _dotx(oh, means)
        pc_ref[...] = pc

        @pl.when(i == 0)
        def _():
            nsq_ref[...] = jnp.zeros_like(nsq_ref)
        nsq_ref[...] += _dotr(oh, pc * pc)

    return pl.pallas_call(
        body,
        grid=(GRID_N,),
        in_specs=[
            pl.BlockSpec((NB, PE), lambda i: (i, 0)),
            pl.BlockSpec((NB, 1), lambda i: (i, 0)),
            pl.BlockSpec((G, PE), lambda i: (0, 0)),
            pl.BlockSpec((G, 8), lambda i: (0, 0)),
        ],
        out_specs=[
            pl.BlockSpec((NB, PE), lambda i: (i, 0)),
            pl.BlockSpec((G, PE), lambda i: (0, 0)),
        ],
        out_shape=[
            jax.ShapeDtypeStruct((NP, PE), F32),
            jax.ShapeDtypeStruct((G, PE), F32),
        ],
    )(pL, gid2d, sums, cnt)


def _k_read3(pc, gid2d, nsq, h4, whp, bhp):
    def body(pc_ref, gid_ref, nsq_ref, h4_ref, w_ref, b_ref, hg_ref):
        i = pl.program_id(0)
        oh = _oh_g(gid_ref)
        rn = lax.rsqrt(jnp.maximum(nsq_ref[...], 1e-30))
        pn = pc_ref[...] * _dotx(oh, rn)
        hpv = jnp.concatenate([h4_ref[...], pn], axis=1)
        hpw = _dot(hpv, w_ref[...]) + b_ref[...]

        @pl.when(i == 0)
        def _():
            hg_ref[...] = jnp.zeros_like(hg_ref)
        hg_ref[...] += _dotr(oh, hpw)

    return pl.pallas_call(
        body,
        grid=(GRID_N,),
        in_specs=[
            pl.BlockSpec((NB, PE), lambda i: (i, 0)),
            pl.BlockSpec((NB, 1), lambda i: (i, 0)),
            pl.BlockSpec((G, PE), lambda i: (0, 0)),
            pl.BlockSpec((NB, HID), lambda i: (i, 0)),
            pl.BlockSpec((HID + PE, HID), lambda i: (0, 0)),
            pl.BlockSpec((1, HID), lambda i: (0, 0)),
        ],
        out_specs=[pl.BlockSpec((G, HID), lambda i: (0, 0))],
        out_shape=[jax.ShapeDtypeStruct((G, HID), F32)],
    )(pc, gid2d, nsq, h4, whp, bhp)


def _k_read4(hgsum, cnt, params):
    def body(hg_ref, cnt_ref, w0, b0, w1, b1, w2, b2, y_ref):
        hg = hg_ref[...] / jnp.maximum(cnt_ref[:, :1], 1.0)
        y = jnp.maximum(_dot(hg, w0[...]) + b0[...], 0.0)
        y = jnp.maximum(_dot(y, w1[...]) + b1[...], 0.0)
        y_ref[...] = _dot(y, w2[...]) + b2[...]

    return pl.pallas_call(
        body,
        out_shape=jax.ShapeDtypeStruct((G, 1), F32),
    )(hgsum, cnt,
      params['mlp0']['w'], params['mlp0']['b'][None, :],
      params['mlp1']['w'], params['mlp1']['b'][None, :],
      params['mlp2']['w'], params['mlp2']['b'][None, :])


def kernel(h, p, e, snorm_n, edge_index, graph_ids, params):
    src = edge_index[0]
    dst = edge_index[1]
    src2d = jnp.pad(src, (0, EP - E), constant_values=N).reshape(EP // 128, 128)
    dst2d = jnp.pad(dst, (0, EP - E), constant_values=N).reshape(EP // 128, 128)
    eids2d = jnp.pad(e, (0, EP - E))[:, None]
    ids2d = jnp.pad(h, (0, NP - N))[:, None]
    p_pad = jnp.pad(p, ((0, NP - N), (0, 0)))
    gid2d = jnp.pad(graph_ids, (0, NP - N), constant_values=G)[:, None]

    hc, pc = _k_embed(ids2d, p_pad, params['emb_h'],
                      params['emb_p']['w'], params['emb_p']['b'][None, :])
    lp0 = params['layers'][0]
    es, b3s = _k_edge_embed(eids2d, params['emb_e'],
                            lp0['B3']['w'], lp0['B3']['b'][None, :])

    for li in range(4):
        lp = params['layers'][li]
        a1h, c1p, b1s, b2s, a2s, c2s = _k_node_pre(hc, pc, lp)
        hatf, ssf = _sc_pass1(src2d, dst2d,
                              b1s.reshape(4 * NP, QW),
                              b2s.reshape(4 * NP, QW),
                              b3s)
        etaf, acch = _sc_pass2(src2d, dst2d, hatf, ssf,
                               a2s.reshape(4 * NP, QW))
        accp = _sc_pass3(src2d, dst2d, etaf, c2s.reshape(4 * NP, QW))
        hc, pc = _k_node_update(hc, pc, a1h, c1p,
                                acch.reshape(4, NP, QW),
                                accp.reshape(4, NP, QW))
        if li < 3:
            nlp = params['layers'][li + 1]
            es, b3s = _k_edge_update(es, hatf,
                                     nlp['B3']['w'], nlp['B3']['b'][None, :])

    pL, sums, cnt = _k_read1(pc, gid2d, params['p_out']['w'],
                             params['p_out']['b'][None, :])
    pcC, nsq = _k_read2(pL, gid2d, sums, cnt)
    hgsum, = _k_read3(pcC, gid2d, nsq, hc, params['Whp']['w'],
                      params['Whp']['b'][None, :])
    return _k_read4(hgsum, cnt, params)


# speedup vs baseline: 2.9924x; 1.4605x over previous
import functools

import jax
import jax.numpy as jnp
from jax import lax
from jax.experimental import pallas as pl
from jax.experimental.pallas import tpu as pltpu
from jax.experimental.pallas import tpu_sc as plsc

N = 50000
E = 800000
G = 128
HID = 64
PE = 16
QW = 16

NP = 50176
EP = 802816
TECS = 16
EP_TEC = EP // TECS
CH = 512
NSUB = CH // 128
NCH = EP_TEC // CH
STR = NP // TECS
ZR = 784

NB = 1792
GRID_N = NP // NB
EB = 3584
GRID_E = EP // EB

_MESH = dict(core_axis_name="c", subcore_axis_name="s")
F32 = jnp.float32
_SCP = pltpu.CompilerParams(use_tc_tiling_on_sc=False)


def _dotr(a, b):
    return lax.dot_general(a, b, (((0,), (0,)), ((), ())),
                           precision=lax.Precision.HIGHEST,
                           preferred_element_type=F32)


def _dot(a, b):
    return jnp.dot(a, b, precision=lax.Precision.HIGHEST,
                   preferred_element_type=F32)


def _dotx(a, b):
    return jnp.dot(a, b, precision=lax.Precision.HIGHEST,
                   preferred_element_type=F32)


def _zero_stripe(zb, sh, s):
    @plsc.parallel_loop(0, ZR, 1, unroll=8)
    def zr(r):
        zb[r, pl.ds(0, 16)] = jnp.zeros((16,), F32)
    for q in range(4):
        pltpu.sync_copy(zb, sh.at[pl.ds(s * STR + q * ZR, ZR)])


def _dump_stripe(sh, acc_h, s, qN):
    for q in range(4):
        pltpu.sync_copy(sh.at[pl.ds(s * STR + q * ZR, ZR)],
                        acc_h.at[pl.ds(qN + s * STR + q * ZR, ZR)])


def _load_adjust_idx(src2d, dst2d, sidx, didx, asrc, adst, r0, qN, need_src):
    pltpu.sync_copy(dst2d.at[pl.ds(r0, NSUB)], didx)
    if need_src:
        pltpu.sync_copy(src2d.at[pl.ds(r0, NSUB)], sidx)
    for j in range(NSUB):
        for t in range(8):
            sl = pl.ds(t * 16, 16)
            if need_src:
                asrc[j, sl] = sidx[j, sl] + qN
            adst[j, sl] = didx[j, sl] + qN


def _sc_pass1(src2d, dst2d, b1f, b2f, b3f):
    @functools.partial(
        pl.kernel,
        out_type=(jax.ShapeDtypeStruct((EP, HID), F32),
                  jax.ShapeDtypeStruct((4 * NP, QW), F32)),
        mesh=plsc.VectorSubcoreMesh(**_MESH),
        compiler_params=_SCP,
        scratch_types=[
            pltpu.VMEM((NSUB, 128), jnp.int32),
            pltpu.VMEM((NSUB, 128), jnp.int32),
            pltpu.VMEM((NSUB, 128), jnp.int32),
            pltpu.VMEM((NSUB, 128), jnp.int32),
            pltpu.VMEM((CH, QW), F32),
            pltpu.VMEM((CH, QW), F32),
            pltpu.VMEM((CH, QW), F32),
            pltpu.VMEM((CH, QW), F32),
            pltpu.VMEM((ZR, QW), F32),
            pltpu.VMEM_SHARED((NP, QW), F32),
            pltpu.SemaphoreType.DMA,
            pltpu.SemaphoreType.DMA,
            pltpu.SemaphoreType.DMA,
        ],
    )
    def k(src_h, dst_h, b1_h, b2_h, b3_h, hat_h, ss_h,
          sidx, didx, asrc, adst, bA, bB, bC, bS, zb, sh, m1, m2, m3):
        c = lax.axis_index("c")
        s = lax.axis_index("s")
        for sub in range(2):
            qid = c * 2 + sub
            qN = qid * NP
            _zero_stripe(zb, sh, s)
            plsc.subcore_barrier()

            def chunk(kk, _):
                base = s * EP_TEC + kk * CH
                r0 = s * (EP_TEC // 128) + kk * NSUB
                _load_adjust_idx(src2d=src_h, dst2d=dst_h, sidx=sidx,
                                 didx=didx, asrc=asrc, adst=adst,
                                 r0=r0, qN=qN, need_src=True)
                cps = []
                for j in range(NSUB):
                    cps.append(pltpu.async_copy(
                        b1_h.at[asrc.at[j]],
                        bA.at[pl.ds(j * 128, 128)], m1))
                    cps.append(pltpu.async_copy(
                        b2_h.at[adst.at[j]],
                        bB.at[pl.ds(j * 128, 128)], m2))
                cps.append(pltpu.async_copy(
                    b3_h.at[pl.ds(base, CH), pl.ds(qid * QW, QW)], bC, m3))
                for cp in cps:
                    cp.wait()

                @plsc.parallel_loop(0, CH, 1, unroll=8)
                def row(r):
                    v = bA[r, pl.ds(0, 16)] + bB[r, pl.ds(0, 16)] \
                        + bC[r, pl.ds(0, 16)]
                    bC[r, pl.ds(0, 16)] = v
                    bS[r, pl.ds(0, 16)] = 1.0 / (1.0 + jnp.exp(-v))
                pltpu.sync_copy(bC,
                                hat_h.at[pl.ds(base, CH), pl.ds(qid * QW, QW)])
                for j in range(NSUB):
                    pltpu.sync_copy(bS.at[pl.ds(j * 128, 128)],
                                    sh.at[didx.at[j]], add=True)
                return 0
            lax.fori_loop(0, NCH, chunk, 0)
            plsc.subcore_barrier()
            _dump_stripe(sh, ss_h, s, qN)
            plsc.subcore_barrier()

    return k(src2d, dst2d, b1f, b2f, b3f)


def _sc_pass2(src2d, dst2d, hatf, ssf, a2f):
    @functools.partial(
        pl.kernel,
        out_type=(jax.ShapeDtypeStruct((EP, HID), F32),
                  jax.ShapeDtypeStruct((4 * NP, QW), F32)),
        mesh=plsc.VectorSubcoreMesh(**_MESH),
        compiler_params=_SCP,
        scratch_types=[
            pltpu.VMEM((NSUB, 128), jnp.int32),
            pltpu.VMEM((NSUB, 128), jnp.int32),
            pltpu.VMEM((NSUB, 128), jnp.int32),
            pltpu.VMEM((NSUB, 128), jnp.int32),
            pltpu.VMEM((CH, QW), F32),
            pltpu.VMEM((CH, QW), F32),
            pltpu.VMEM((CH, QW), F32),
            pltpu.VMEM((CH, QW), F32),
            pltpu.VMEM((ZR, QW), F32),
            pltpu.VMEM_SHARED((NP, QW), F32),
            pltpu.SemaphoreType.DMA,
            pltpu.SemaphoreType.DMA,
            pltpu.SemaphoreType.DMA,
        ],
    )
    def k(src_h, dst_h, hat_h, ssn_h, a2_h, eta_h, acc_h,
          sidx, didx, asrc, adst, bA, bB, bC, bS, zb, sh, m1, m2, m3):
        c = lax.axis_index("c")
        s = lax.axis_index("s")
        for sub in range(2):
            qid = c * 2 + sub
            qN = qid * NP
            _zero_stripe(zb, sh, s)
            plsc.subcore_barrier()

            def chunk(kk, _):
                base = s * EP_TEC + kk * CH
                r0 = s * (EP_TEC // 128) + kk * NSUB
                _load_adjust_idx(src2d=src_h, dst2d=dst_h, sidx=sidx,
                                 didx=didx, asrc=asrc, adst=adst,
                                 r0=r0, qN=qN, need_src=True)
                cps = []
                for j in range(NSUB):
                    cps.append(pltpu.async_copy(
                        a2_h.at[asrc.at[j]],
                        bA.at[pl.ds(j * 128, 128)], m1))
                    cps.append(pltpu.async_copy(
                        ssn_h.at[adst.at[j]],
                        bB.at[pl.ds(j * 128, 128)], m2))
                cps.append(pltpu.async_copy(
                    hat_h.at[pl.ds(base, CH), pl.ds(qid * QW, QW)], bC, m3))
                for cp in cps:
                    cp.wait()

                @plsc.parallel_loop(0, CH, 1, unroll=8)
                def row(r):
                    v = bC[r, pl.ds(0, 16)]
                    sg = 1.0 / (1.0 + jnp.exp(-v))
                    eta = sg / (bB[r, pl.ds(0, 16)] + 1e-6)
                    bC[r, pl.ds(0, 16)] = eta
                    bS[r, pl.ds(0, 16)] = eta * bA[r, pl.ds(0, 16)]
                pltpu.sync_copy(bC,
                                eta_h.at[pl.ds(base, CH), pl.ds(qid * QW, QW)])
                for j in range(NSUB):
                    pltpu.sync_copy(bS.at[pl.ds(j * 128, 128)],
                                    sh.at[didx.at[j]], add=True)
                return 0
            lax.fori_loop(0, NCH, chunk, 0)
            plsc.subcore_barrier()
            _dump_stripe(sh, acc_h, s, qN)
            plsc.subcore_barrier()

    return k(src2d, dst2d, hatf, ssf, a2f)


def _sc_pass3(src2d, dst2d, etaf, c2f):
    @functools.partial(
        pl.kernel,
        out_type=jax.ShapeDtypeStruct((4 * NP, QW), F32),
        mesh=plsc.VectorSubcoreMesh(**_MESH),
        compiler_params=_SCP,
        scratch_types=[
            pltpu.VMEM((NSUB, 128), jnp.int32),
            pltpu.VMEM((NSUB, 128), jnp.int32),
            pltpu.VMEM((NSUB, 128), jnp.int32),
            pltpu.VMEM((NSUB, 128), jnp.int32),
            pltpu.VMEM((CH, QW), F32),
            pltpu.VMEM((CH, QW), F32),
            pltpu.VMEM((CH, QW), F32),
            pltpu.VMEM((ZR, QW), F32),
            pltpu.VMEM_SHARED((NP, QW), F32),
            pltpu.SemaphoreType.DMA,
            pltpu.SemaphoreType.DMA,
        ],
    )
    def k(src_h, dst_h, eta_h, c2_h, acc_h,
          sidx, didx, asrc, adst, bA, bC, bS, zb, sh, m1, m3):
        c = lax.axis_index("c")
        s = lax.axis_index("s")
        for sub in range(2):
            qid = c * 2 + sub
            qN = qid * NP
            _zero_stripe(zb, sh, s)
            plsc.subcore_barrier()

            def chunk(kk, _):
                base = s * EP_TEC + kk * CH
                r0 = s * (EP_TEC // 128) + kk * NSUB
                _load_adjust_idx(src2d=src_h, dst2d=dst_h, sidx=sidx,
                                 didx=didx, asrc=asrc, adst=adst,
                                 r0=r0, qN=qN, need_src=True)
                cps = []
                for j in range(NSUB):
                    cps.append(pltpu.async_copy(
                        c2_h.at[asrc.at[j]],
                        bA.at[pl.ds(j * 128, 128)], m1))
                cps.append(pltpu.async_copy(
                    eta_h.at[pl.ds(base, CH), pl.ds(qid * QW, QW)], bC, m3))
                for cp in cps:
                    cp.wait()

                @plsc.parallel_loop(0, CH, 1, unroll=8)
                def row(r):
                    bS[r, pl.ds(0, 16)] = bC[r, pl.ds(0, 16)] \
                        * bA[r, pl.ds(0, 16)]
                for j in range(NSUB):
                    pltpu.sync_copy(bS.at[pl.ds(j * 128, 128)],
                                    sh.at[didx.at[j]], add=True)
                return 0
            lax.fori_loop(0, NCH, chunk, 0)
            plsc.subcore_barrier()
            _dump_stripe(sh, acc_h, s, qN)
            plsc.subcore_barrier()

    return k(src2d, dst2d, etaf, c2f)


def _split4(ref, t):
    for q in range(4):
        ref[q, :, :] = t[:, q * QW:(q + 1) * QW]


def _k_embed(ids2d, p_pad, emb_h, wp, bp):
    def body(ids_ref, p_ref, eh_ref, wp_ref, bp_ref, h0_ref, p0_ref):
        oh = (ids_ref[...] ==
              lax.broadcasted_iota(jnp.int32, (1, 28), 1)).astype(F32)
        h0_ref[...] = _dotx(oh, eh_ref[...])
        p0_ref[...] = _dot(p_ref[...], wp_ref[...]) + bp_ref[...]

    return pl.pallas_call(
        body,
        grid=(GRID_N,),
        in_specs=[
            pl.BlockSpec((NB, 1), lambda i: (i, 0)),
            pl.BlockSpec((NB, PE), lambda i: (i, 0)),
            pl.BlockSpec((28, HID), lambda i: (0, 0)),
            pl.BlockSpec((PE, HID), lambda i: (0, 0)),
            pl.BlockSpec((1, HID), lambda i: (0, 0)),
        ],
        out_specs=[
            pl.BlockSpec((NB, HID), lambda i: (i, 0)),
            pl.BlockSpec((NB, HID), lambda i: (i, 0)),
        ],
        out_shape=[
            jax.ShapeDtypeStruct((NP, HID), F32),
            jax.ShapeDtypeStruct((NP, HID), F32),
        ],
    )(ids2d, p_pad, emb_h, wp, bp)


def _k_node_pre(h, p, lp):
    def body(h_ref, p_ref, wa1, ba1, wa2, ba2, wb1, bb1, wb2, bb2,
             wc1, bc1, wc2, bc2, a1_ref, c1_ref, b1s, b2s, a2s, c2s):
        hp = jnp.concatenate([h_ref[...], p_ref[...]], axis=1)
        a1_ref[...] = _dot(hp, wa1[...]) + ba1[...]
        c1_ref[...] = _dot(p_ref[...], wc1[...]) + bc1[...]
        _split4(b1s, _dot(hp, wb1[...]) + bb1[...])
        _split4(b2s, _dot(hp, wb2[...]) + bb2[...])
        _split4(a2s, _dot(hp, wa2[...]) + ba2[...])
        _split4(c2s, _dot(p_ref[...], wc2[...]) + bc2[...])

    wspec = lambda shp: pl.BlockSpec(shp, lambda i: (0, 0))
    nspec = pl.BlockSpec((NB, HID), lambda i: (i, 0))
    sspec = pl.BlockSpec((4, NB, QW), lambda i: (0, i, 0))
    return pl.pallas_call(
        body,
        grid=(GRID_N,),
        in_specs=[nspec, nspec,
                  wspec((2 * HID, HID)), wspec((1, HID)),
                  wspec((2 * HID, HID)), wspec((1, HID)),
                  wspec((2 * HID, HID)), wspec((1, HID)),
                  wspec((2 * HID, HID)), wspec((1, HID)),
                  wspec((HID, HID)), wspec((1, HID)),
                  wspec((HID, HID)), wspec((1, HID))],
        out_specs=[nspec, nspec, sspec, sspec, sspec, sspec],
        out_shape=[
            jax.ShapeDtypeStruct((NP, HID), F32),
            jax.ShapeDtypeStruct((NP, HID), F32),
            jax.ShapeDtypeStruct((4, NP, QW), F32),
            jax.ShapeDtypeStruct((4, NP, QW), F32),
            jax.ShapeDtypeStruct((4, NP, QW), F32),
            jax.ShapeDtypeStruct((4, NP, QW), F32),
        ],
    )(h, p,
      lp['A1']['w'], lp['A1']['b'][None, :],
      lp['A2']['w'], lp['A2']['b'][None, :],
      lp['B1']['w'], lp['B1']['b'][None, :],
      lp['B2']['w'], lp['B2']['b'][None, :],
      lp['C1']['w'], lp['C1']['b'][None, :],
      lp['C2']['w'], lp['C2']['b'][None, :])


def _k_edge_embed(eids2d, emb_e, w3, b3):
    def body(ids_ref, ee_ref, w3_ref, b3_ref, es_ref, b3s_ref):
        oh = (ids_ref[...] ==
              lax.broadcasted_iota(jnp.int32, (1, 4), 1)).astype(F32)
        e1 = _dotx(oh, ee_ref[...])
        es_ref[...] = e1
        b3s_ref[...] = _dot(e1, w3_ref[...]) + b3_ref[...]

    espec = pl.BlockSpec((EB, HID), lambda i: (i, 0))
    return pl.pallas_call(
        body,
        grid=(GRID_E,),
        in_specs=[
            pl.BlockSpec((EB, 1), lambda i: (i, 0)),
            pl.BlockSpec((4, HID), lambda i: (0, 0)),
            pl.BlockSpec((HID, HID), lambda i: (0, 0)),
            pl.BlockSpec((1, HID), lambda i: (0, 0)),
        ],
        out_specs=[espec, espec],
        out_shape=[
            jax.ShapeDtypeStruct((EP, HID), F32),
            jax.ShapeDtypeStruct((EP, HID), F32),
        ],
    )(eids2d, emb_e, w3, b3)


def _k_edge_update(es, hats, w3, b3):
    def body(es_ref, ht_ref, w3_ref, b3_ref, es2_ref, b3s_ref):
        e2 = es_ref[...] + jnp.maximum(ht_ref[...], 0.0)
        es2_ref[...] = e2
        b3s_ref[...] = _dot(e2, w3_ref[...]) + b3_ref[...]

    espec = pl.BlockSpec((EB, HID), lambda i: (i, 0))
    return pl.pallas_call(
        body,
        grid=(GRID_E,),
        in_specs=[espec, espec,
                  pl.BlockSpec((HID, HID), lambda i: (0, 0)),
                  pl.BlockSpec((1, HID), lambda i: (0, 0))],
        out_specs=[espec, espec],
        out_shape=[
            jax.ShapeDtypeStruct((EP, HID), F32),
            jax.ShapeDtypeStruct((EP, HID), F32),
        ],
    )(es, hats, w3, b3)


def _k_node_update(h, p, a1h, c1p, acchs, accps):
    def body(h_ref, p_ref, a1_ref, c1_ref, ah_ref, ap_ref, h2_ref, p2_ref):
        acch = jnp.concatenate([ah_ref[q, :, :] for q in range(4)], axis=1)
        accp = jnp.concatenate([ap_ref[q, :, :] for q in range(4)], axis=1)
        h2_ref[...] = h_ref[...] + jnp.maximum(a1_ref[...] + acch, 0.0)
        p2_ref[...] = p_ref[...] + jnp.tanh(c1_ref[...] + accp)

    nspec = pl.BlockSpec((NB, HID), lambda i: (i, 0))
    sspec = pl.BlockSpec((4, NB, QW), lambda i: (0, i, 0))
    return pl.pallas_call(
        body,
        grid=(GRID_N,),
        in_specs=[nspec, nspec, nspec, nspec, sspec, sspec],
        out_specs=[nspec, nspec],
        out_shape=[
            jax.ShapeDtypeStruct((NP, HID), F32),
            jax.ShapeDtypeStruct((NP, HID), F32),
        ],
    )(h, p, a1h, c1p, acchs, accps)


def _oh_g(gid_ref):
    return (gid_ref[...] ==
            lax.broadcasted_iota(jnp.int32, (1, G), 1)).astype(F32)


def _k_read1(p4, gid2d, wpo, bpo):
    def body(p4_ref, gid_ref, w_ref, b_ref, pl_ref, sums_ref, cnt_ref):
        i = pl.program_id(0)
        oh = _oh_g(gid_ref)
        plv = _dot(p4_ref[...], w_ref[...]) + b_ref[...]
        pl_ref[...] = plv

        @pl.when(i == 0)
        def _():
            sums_ref[...] = jnp.zeros_like(sums_ref)
            cnt_ref[...] = jnp.zeros_like(cnt_ref)
        sums_ref[...] += _dotr(oh, plv)
        cnt_ref[...] += _dotr(oh, jnp.ones((NB, 8), F32))

    return pl.pallas_call(
        body,
        grid=(GRID_N,),
        in_specs=[
            pl.BlockSpec((NB, HID), lambda i: (i, 0)),
            pl.BlockSpec((NB, 1), lambda i: (i, 0)),
            pl.BlockSpec((HID, PE), lambda i: (0, 0)),
            pl.BlockSpec((1, PE), lambda i: (0, 0)),
        ],
        out_specs=[
            pl.BlockSpec((NB, PE), lambda i: (i, 0)),
            pl.BlockSpec((G, PE), lambda i: (0, 0)),
            pl.BlockSpec((G, 8), lambda i: (0, 0)),
        ],
        out_shape=[
            jax.ShapeDtypeStruct((NP, PE), F32),
            jax.ShapeDtypeStruct((G, PE), F32),
            jax.ShapeDtypeStruct((G, 8), F32),
        ],
    )(p4, gid2d, wpo, bpo)


def _k_read2(pL, gid2d, sums, cnt):
    def body(pl_ref, gid_ref, sums_ref, cnt_ref, pc_ref, nsq_ref):
        i = pl.program_id(0)
        oh = _oh_g(gid_ref)
        means = sums_ref[...] / jnp.maximum(cnt_ref[:, :1], 1.0)
        pc = pl_ref[...] - _dotx(oh, means)
        pc_ref[...] = pc

        @pl.when(i == 0)
        def _():
            nsq_ref[...] = jnp.zeros_like(nsq_ref)
        nsq_ref[...] += _dotr(oh, pc * pc)

    return pl.pallas_call(
        body,
        grid=(GRID_N,),
        in_specs=[
            pl.BlockSpec((NB, PE), lambda i: (i, 0)),
            pl.BlockSpec((NB, 1), lambda i: (i, 0)),
            pl.BlockSpec((G, PE), lambda i: (0, 0)),
            pl.BlockSpec((G, 8), lambda i: (0, 0)),
        ],
        out_specs=[
            pl.BlockSpec((NB, PE), lambda i: (i, 0)),
            pl.BlockSpec((G, PE), lambda i: (0, 0)),
        ],
        out_shape=[
            jax.ShapeDtypeStruct((NP, PE), F32),
            jax.ShapeDtypeStruct((G, PE), F32),
        ],
    )(pL, gid2d, sums, cnt)


def _k_read3(pc, gid2d, nsq, h4, whp, bhp):
    def body(pc_ref, gid_ref, nsq_ref, h4_ref, w_ref, b_ref, hg_ref):
        i = pl.program_id(0)
        oh = _oh_g(gid_ref)
        rn = lax.rsqrt(jnp.maximum(nsq_ref[...], 1e-30))
        pn = pc_ref[...] * _dotx(oh, rn)
        hpv = jnp.concatenate([h4_ref[...], pn], axis=1)
        hpw = _dot(hpv, w_ref[...]) + b_ref[...]

        @pl.when(i == 0)
        def _():
            hg_ref[...] = jnp.zeros_like(hg_ref)
        hg_ref[...] += _dotr(oh, hpw)

    return pl.pallas_call(
        body,
        grid=(GRID_N,),
        in_specs=[
            pl.BlockSpec((NB, PE), lambda i: (i, 0)),
            pl.BlockSpec((NB, 1), lambda i: (i, 0)),
            pl.BlockSpec((G, PE), lambda i: (0, 0)),
            pl.BlockSpec((NB, HID), lambda i: (i, 0)),
            pl.BlockSpec((HID + PE, HID), lambda i: (0, 0)),
            pl.BlockSpec((1, HID), lambda i: (0, 0)),
        ],
        out_specs=[pl.BlockSpec((G, HID), lambda i: (0, 0))],
        out_shape=[jax.ShapeDtypeStruct((G, HID), F32)],
    )(pc, gid2d, nsq, h4, whp, bhp)


def _k_read4(hgsum, cnt, params):
    def body(hg_ref, cnt_ref, w0, b0, w1, b1, w2, b2, y_ref):
        hg = hg_ref[...] / jnp.maximum(cnt_ref[:, :1], 1.0)
        y = jnp.maximum(_dot(hg, w0[...]) + b0[...], 0.0)
        y = jnp.maximum(_dot(y, w1[...]) + b1[...], 0.0)
        y_ref[...] = _dot(y, w2[...]) + b2[...]

    return pl.pallas_call(
        body,
        out_shape=jax.ShapeDtypeStruct((G, 1), F32),
    )(hgsum, cnt,
      params['mlp0']['w'], params['mlp0']['b'][None, :],
      params['mlp1']['w'], params['mlp1']['b'][None, :],
      params['mlp2']['w'], params['mlp2']['b'][None, :])


def kernel(h, p, e, snorm_n, edge_index, graph_ids, params):
    src = edge_index[0]
    dst = edge_index[1]
    src2d = jnp.pad(src, (0, EP - E), constant_values=N).reshape(EP // 128, 128)
    dst2d = jnp.pad(dst, (0, EP - E), constant_values=N).reshape(EP // 128, 128)
    eids2d = jnp.pad(e, (0, EP - E))[:, None]
    ids2d = jnp.pad(h, (0, NP - N))[:, None]
    p_pad = jnp.pad(p, ((0, NP - N), (0, 0)))
    gid2d = jnp.pad(graph_ids, (0, NP - N), constant_values=G)[:, None]

    hc, pc = _k_embed(ids2d, p_pad, params['emb_h'],
                      params['emb_p']['w'], params['emb_p']['b'][None, :])
    lp0 = params['layers'][0]
    es, b3s = _k_edge_embed(eids2d, params['emb_e'],
                            lp0['B3']['w'], lp0['B3']['b'][None, :])

    for li in range(4):
        lp = params['layers'][li]
        a1h, c1p, b1s, b2s, a2s, c2s = _k_node_pre(hc, pc, lp)
        hatf, ssf = _sc_pass1(src2d, dst2d,
                              b1s.reshape(4 * NP, QW),
                              b2s.reshape(4 * NP, QW),
                              b3s)
        etaf, acch = _sc_pass2(src2d, dst2d, hatf, ssf,
                               a2s.reshape(4 * NP, QW))
        accp = _sc_pass3(src2d, dst2d, etaf, c2s.reshape(4 * NP, QW))
        hc, pc = _k_node_update(hc, pc, a1h, c1p,
                                acch.reshape(4, NP, QW),
                                accp.reshape(4, NP, QW))
        if li < 3:
            nlp = params['layers'][li + 1]
            es, b3s = _k_edge_update(es, hatf,
                                     nlp['B3']['w'], nlp['B3']['b'][None, :])

    pL, sums, cnt = _k_read1(pc, gid2d, params['p_out']['w'],
                             params['p_out']['b'][None, :])
    pcC, nsq = _k_read2(pL, gid2d, sums, cnt)
    hgsum, = _k_read3(pcC, gid2d, nsq, hc, params['Whp']['w'],
                      params['Whp']['b'][None, :])
    return _k_read4(hgsum, cnt, params)


# speedup vs baseline: 4.1114x; 1.3739x over previous
import functools

import jax
import jax.numpy as jnp
from jax import lax
from jax.experimental import pallas as pl
from jax.experimental.pallas import tpu as pltpu
from jax.experimental.pallas import tpu_sc as plsc

N = 50000
E = 800000
G = 128
HID = 64
PE = 16
QW = 16

NP = 50176
EP = 802816
TECS = 16
EP_TEC = EP // TECS
CH = 256
NSUB = CH // 128
NCH = EP_TEC // CH
STR = NP // TECS
ZR = 784

NB = 1792
GRID_N = NP // NB
EB = 3584
GRID_E = EP // EB

_MESH = dict(core_axis_name="c", subcore_axis_name="s")
F32 = jnp.float32
_SCP = pltpu.CompilerParams(use_tc_tiling_on_sc=False)


def _dotr(a, b):
    return lax.dot_general(a, b, (((0,), (0,)), ((), ())),
                           precision=lax.Precision.HIGHEST,
                           preferred_element_type=F32)


def _dot(a, b):
    return jnp.dot(a, b, precision=lax.Precision.HIGHEST,
                   preferred_element_type=F32)


def _dotx(a, b):
    return jnp.dot(a, b, precision=lax.Precision.HIGHEST,
                   preferred_element_type=F32)


def _zero_stripe(zb, sh, s):
    @plsc.parallel_loop(0, ZR, 1, unroll=8)
    def zr(r):
        zb[r, pl.ds(0, 16)] = jnp.zeros((16,), F32)
    for q in range(4):
        pltpu.sync_copy(zb, sh.at[pl.ds(s * STR + q * ZR, ZR)])


def _dump_stripe(sh, acc_h, s, qN):
    for q in range(4):
        pltpu.sync_copy(sh.at[pl.ds(s * STR + q * ZR, ZR)],
                        acc_h.at[pl.ds(qN + s * STR + q * ZR, ZR)])


def _load_adjust_idx(src2d, dst2d, sidx, didx, asrc, adst, r0, qN, need_src):
    pltpu.sync_copy(dst2d.at[pl.ds(r0, NSUB)], didx)
    if need_src:
        pltpu.sync_copy(src2d.at[pl.ds(r0, NSUB)], sidx)
    for j in range(NSUB):
        for t in range(8):
            sl = pl.ds(t * 16, 16)
            if need_src:
                asrc[j, sl] = sidx[j, sl] + qN
            adst[j, sl] = didx[j, sl] + qN


def _sweep(s, qid, qN, src_h, dst_h, t1_h, t2_h, lin_h, out_h, sh,
           SIDX, DIDX, ASRC, AGD, DSC, BA, BB, BC, BS,
           MI, MA, MB, MC, MH, MS, compute):
    two = t2_h is not None

    def issue_idx(sl, kkv):
        r0 = s * (EP_TEC // 128) + kkv * NSUB
        pltpu.async_copy(src_h.at[pl.ds(r0, NSUB)], SIDX[sl], MI[sl])
        pltpu.async_copy(dst_h.at[pl.ds(r0, NSUB)], DIDX[sl], MI[sl])

    def wait_idx(sl):
        pltpu.make_async_copy(src_h.at[pl.ds(0, NSUB)], SIDX[sl],
                              MI[sl]).wait()
        pltpu.make_async_copy(dst_h.at[pl.ds(0, NSUB)], DIDX[sl],
                              MI[sl]).wait()

    def adjust(sl):
        for j in range(NSUB):
            for t in range(8):
                c_ = pl.ds(t * 16, 16)
                ASRC[sl][j, c_] = SIDX[sl][j, c_] + qN
                if two:
                    AGD[sl][j, c_] = DIDX[sl][j, c_] + qN
                DSC[sl][j, c_] = DIDX[sl][j, c_]

    def fire(sl, kkv):
        base = s * EP_TEC + kkv * CH
        for j in range(NSUB):
            pltpu.async_copy(t1_h.at[ASRC[sl].at[j]],
                             BA[sl].at[pl.ds(j * 128, 128)], MA[sl])
        if two:
            for j in range(NSUB):
                pltpu.async_copy(t2_h.at[AGD[sl].at[j]],
                                 BB[sl].at[pl.ds(j * 128, 128)], MB[sl])
        pltpu.async_copy(lin_h.at[pl.ds(base, CH), pl.ds(qid * QW, QW)],
                         BC[sl], MC[sl])

    def wait_fire(sl):
        for j in range(NSUB):
            pltpu.make_async_copy(t1_h.at[ASRC[sl].at[j]],
                                  BA[sl].at[pl.ds(j * 128, 128)],
                                  MA[sl]).wait()
        if two:
            for j in range(NSUB):
                pltpu.make_async_copy(t2_h.at[AGD[sl].at[j]],
                                      BB[sl].at[pl.ds(j * 128, 128)],
                                      MB[sl]).wait()
        pltpu.make_async_copy(lin_h.at[pl.ds(0, CH), pl.ds(0, QW)],
                              BC[sl], MC[sl]).wait()

    def store(sl, kkv):
        base = s * EP_TEC + kkv * CH
        if out_h is not None:
            pltpu.async_copy(
                BC[sl], out_h.at[pl.ds(base, CH), pl.ds(qid * QW, QW)],
                MH[sl])
        for j in range(NSUB):
            pltpu.make_async_copy(BS[sl].at[pl.ds(j * 128, 128)],
                                  sh.at[DSC[sl].at[j]],
                                  MS[sl]).start(add=True)

    def drain_out(sl):
        if out_h is not None:
            pltpu.make_async_copy(
                BC[sl], out_h.at[pl.ds(0, CH), pl.ds(0, QW)],
                MH[sl]).wait()
        for j in range(NSUB):
            pltpu.make_async_copy(BS[sl].at[pl.ds(j * 128, 128)],
                                  sh.at[DSC[sl].at[j]], MS[sl]).wait()

    issue_idx(0, 0)
    issue_idx(1, 1)
    wait_idx(0)
    adjust(0)
    fire(0, 0)
    issue_idx(0, 2)

    def gl(t, _):
        k0 = t * 2
        wait_idx(1)

        @pl.when(t > 0)
        def _():
            drain_out(1)
        adjust(1)
        fire(1, k0 + 1)

        @pl.when(k0 + 3 < NCH)
        def _():
            issue_idx(1, k0 + 3)
        wait_fire(0)
        compute(0)
        store(0, k0)

        @pl.when(k0 + 2 < NCH)
        def _():
            wait_idx(0)
            drain_out(0)
            adjust(0)
            fire(0, k0 + 2)

        @pl.when(k0 + 4 < NCH)
        def _():
            issue_idx(0, k0 + 4)
        wait_fire(1)
        compute(1)
        store(1, k0 + 1)
        return 0
    lax.fori_loop(0, NCH // 2, gl, 0)
    drain_out(0)
    drain_out(1)


_IDX2 = [pltpu.VMEM((NSUB, 128), jnp.int32)] * 2
_BUF2 = [pltpu.VMEM((CH, QW), F32)] * 2
_SEM2 = [pltpu.SemaphoreType.DMA] * 2
_SCRATCH12 = (_IDX2 * 5 + _BUF2 * 4
              + [pltpu.VMEM((ZR, QW), F32),
                 pltpu.VMEM_SHARED((NP, QW), F32)]
              + _SEM2 * 6)
_SCRATCH3 = (_IDX2 * 4 + _BUF2 * 3
             + [pltpu.VMEM((ZR, QW), F32),
                pltpu.VMEM_SHARED((NP, QW), F32)]
             + _SEM2 * 4)


def _sc_pass12(src2d, dst2d, t1, t2, lin, which):
    @functools.partial(
        pl.kernel,
        out_type=(jax.ShapeDtypeStruct((EP, HID), F32),
                  jax.ShapeDtypeStruct((4 * NP, QW), F32)),
        mesh=plsc.VectorSubcoreMesh(**_MESH),
        compiler_params=_SCP,
        scratch_types=_SCRATCH12,
        name=f"sc_pass{which}",
    )
    def k(src_h, dst_h, t1_h, t2_h, lin_h, out_h, acc_h, *sc):
        (si0, si1, di0, di1, as0, as1, ag0, ag1, dc0, dc1,
         ba0, ba1, bb0, bb1, bc0, bc1, bs0, bs1, zb, sh,
         mi0, mi1, ma0, ma1, mb0, mb1, mc0, mc1, mh0, mh1,
         ms0, ms1) = sc
        BA, BB, BC, BS = (ba0, ba1), (bb0, bb1), (bc0, bc1), (bs0, bs1)

        def compute(sl):
            bA, bB, bC, bS = BA[sl], BB[sl], BC[sl], BS[sl]
            if which == 1:
                @plsc.parallel_loop(0, CH, 1, unroll=8)
                def row(r):
                    v = bA[r, pl.ds(0, 16)] + bB[r, pl.ds(0, 16)] \
                        + bC[r, pl.ds(0, 16)]
                    bC[r, pl.ds(0, 16)] = v
                    bS[r, pl.ds(0, 16)] = 1.0 / (1.0 + jnp.exp(-v))
            else:
                @plsc.parallel_loop(0, CH, 1, unroll=8)
                def row(r):
                    v = bC[r, pl.ds(0, 16)]
                    sg = 1.0 / (1.0 + jnp.exp(-v))
                    eta = sg / (bB[r, pl.ds(0, 16)] + 1e-6)
                    bC[r, pl.ds(0, 16)] = eta
                    bS[r, pl.ds(0, 16)] = eta * bA[r, pl.ds(0, 16)]

        c = lax.axis_index("c")
        s = lax.axis_index("s")
        for sub in range(2):
            qid = c * 2 + sub
            qN = qid * NP
            _zero_stripe(zb, sh, s)
            plsc.subcore_barrier()
            _sweep(s, qid, qN, src_h, dst_h, t1_h, t2_h, lin_h, out_h, sh,
                   (si0, si1), (di0, di1), (as0, as1), (ag0, ag1),
                   (dc0, dc1), BA, BB, BC, BS,
                   (mi0, mi1), (ma0, ma1), (mb0, mb1), (mc0, mc1),
                   (mh0, mh1), (ms0, ms1), compute)
            plsc.subcore_barrier()
            _dump_stripe(sh, acc_h, s, qN)
            plsc.subcore_barrier()

    return k(src2d, dst2d, t1, t2, lin)


def _sc_pass1(src2d, dst2d, b1f, b2f, b3f):
    return _sc_pass12(src2d, dst2d, b1f, b2f, b3f, 1)


def _sc_pass2(src2d, dst2d, hatf, ssf, a2f):
    return _sc_pass12(src2d, dst2d, a2f, ssf, hatf, 2)


def _sc_pass3(src2d, dst2d, etaf, c2f):
    @functools.partial(
        pl.kernel,
        out_type=jax.ShapeDtypeStruct((4 * NP, QW), F32),
        mesh=plsc.VectorSubcoreMesh(**_MESH),
        compiler_params=_SCP,
        scratch_types=_SCRATCH3,
        name="sc_pass3",
    )
    def k(src_h, dst_h, eta_h, c2_h, acc_h, *sc):
        (si0, si1, di0, di1, as0, as1, dc0, dc1,
         ba0, ba1, bc0, bc1, bs0, bs1, zb, sh,
         mi0, mi1, ma0, ma1, mc0, mc1, ms0, ms1) = sc
        BA, BC, BS = (ba0, ba1), (bc0, bc1), (bs0, bs1)

        def compute(sl):
            bA, bC, bS = BA[sl], BC[sl], BS[sl]

            @plsc.parallel_loop(0, CH, 1, unroll=8)
            def row(r):
                bS[r, pl.ds(0, 16)] = bC[r, pl.ds(0, 16)] \
                    * bA[r, pl.ds(0, 16)]

        c = lax.axis_index("c")
        s = lax.axis_index("s")
        for sub in range(2):
            qid = c * 2 + sub
            qN = qid * NP
            _zero_stripe(zb, sh, s)
            plsc.subcore_barrier()
            _sweep(s, qid, qN, src_h, dst_h, c2_h, None, eta_h, None, sh,
                   (si0, si1), (di0, di1), (as0, as1), None,
                   (dc0, dc1), BA, None, BC, BS,
                   (mi0, mi1), (ma0, ma1), None, (mc0, mc1),
                   None, (ms0, ms1), compute)
            plsc.subcore_barrier()
            _dump_stripe(sh, acc_h, s, qN)
            plsc.subcore_barrier()

    return k(src2d, dst2d, etaf, c2f)


def _split4(ref, t):
    for q in range(4):
        ref[q, :, :] = t[:, q * QW:(q + 1) * QW]


def _k_embed(ids2d, p_pad, emb_h, wp, bp):
    def body(ids_ref, p_ref, eh_ref, wp_ref, bp_ref, h0_ref, p0_ref):
        oh = (ids_ref[...] ==
              lax.broadcasted_iota(jnp.int32, (1, 28), 1)).astype(F32)
        h0_ref[...] = _dotx(oh, eh_ref[...])
        p0_ref[...] = _dot(p_ref[...], wp_ref[...]) + bp_ref[...]

    return pl.pallas_call(
        body,
        grid=(GRID_N,),
        in_specs=[
            pl.BlockSpec((NB, 1), lambda i: (i, 0)),
            pl.BlockSpec((NB, PE), lambda i: (i, 0)),
            pl.BlockSpec((28, HID), lambda i: (0, 0)),
            pl.BlockSpec((PE, HID), lambda i: (0, 0)),
            pl.BlockSpec((1, HID), lambda i: (0, 0)),
        ],
        out_specs=[
            pl.BlockSpec((NB, HID), lambda i: (i, 0)),
            pl.BlockSpec((NB, HID), lambda i: (i, 0)),
        ],
        out_shape=[
            jax.ShapeDtypeStruct((NP, HID), F32),
            jax.ShapeDtypeStruct((NP, HID), F32),
        ],
    )(ids2d, p_pad, emb_h, wp, bp)


def _k_node_pre(h, p, lp):
    def body(h_ref, p_ref, wa1, ba1, wa2, ba2, wb1, bb1, wb2, bb2,
             wc1, bc1, wc2, bc2, a1_ref, c1_ref, b1s, b2s, a2s, c2s):
        hp = jnp.concatenate([h_ref[...], p_ref[...]], axis=1)
        a1_ref[...] = _dot(hp, wa1[...]) + ba1[...]
        c1_ref[...] = _dot(p_ref[...], wc1[...]) + bc1[...]
        _split4(b1s, _dot(hp, wb1[...]) + bb1[...])
        _split4(b2s, _dot(hp, wb2[...]) + bb2[...])
        _split4(a2s, _dot(hp, wa2[...]) + ba2[...])
        _split4(c2s, _dot(p_ref[...], wc2[...]) + bc2[...])

    wspec = lambda shp: pl.BlockSpec(shp, lambda i: (0, 0))
    nspec = pl.BlockSpec((NB, HID), lambda i: (i, 0))
    sspec = pl.BlockSpec((4, NB, QW), lambda i: (0, i, 0))
    return pl.pallas_call(
        body,
        grid=(GRID_N,),
        in_specs=[nspec, nspec,
                  wspec((2 * HID, HID)), wspec((1, HID)),
                  wspec((2 * HID, HID)), wspec((1, HID)),
                  wspec((2 * HID, HID)), wspec((1, HID)),
                  wspec((2 * HID, HID)), wspec((1, HID)),
                  wspec((HID, HID)), wspec((1, HID)),
                  wspec((HID, HID)), wspec((1, HID))],
        out_specs=[nspec, nspec, sspec, sspec, sspec, sspec],
        out_shape=[
            jax.ShapeDtypeStruct((NP, HID), F32),
            jax.ShapeDtypeStruct((NP, HID), F32),
            jax.ShapeDtypeStruct((4, NP, QW), F32),
            jax.ShapeDtypeStruct((4, NP, QW), F32),
            jax.ShapeDtypeStruct((4, NP, QW), F32),
            jax.ShapeDtypeStruct((4, NP, QW), F32),
        ],
    )(h, p,
      lp['A1']['w'], lp['A1']['b'][None, :],
      lp['A2']['w'], lp['A2']['b'][None, :],
      lp['B1']['w'], lp['B1']['b'][None, :],
      lp['B2']['w'], lp['B2']['b'][None, :],
      lp['C1']['w'], lp['C1']['b'][None, :],
      lp['C2']['w'], lp['C2']['b'][None, :])


def _k_edge_embed(eids2d, emb_e, w3, b3):
    def body(ids_ref, ee_ref, w3_ref, b3_ref, es_ref, b3s_ref):
        oh = (ids_ref[...] ==
              lax.broadcasted_iota(jnp.int32, (1, 4), 1)).astype(F32)
        e1 = _dotx(oh, ee_ref[...])
        es_ref[...] = e1
        b3s_ref[...] = _dot(e1, w3_ref[...]) + b3_ref[...]

    espec = pl.BlockSpec((EB, HID), lambda i: (i, 0))
    return pl.pallas_call(
        body,
        grid=(GRID_E,),
        in_specs=[
            pl.BlockSpec((EB, 1), lambda i: (i, 0)),
            pl.BlockSpec((4, HID), lambda i: (0, 0)),
            pl.BlockSpec((HID, HID), lambda i: (0, 0)),
            pl.BlockSpec((1, HID), lambda i: (0, 0)),
        ],
        out_specs=[espec, espec],
        out_shape=[
            jax.ShapeDtypeStruct((EP, HID), F32),
            jax.ShapeDtypeStruct((EP, HID), F32),
        ],
    )(eids2d, emb_e, w3, b3)


def _k_edge_update(es, hats, w3, b3):
    def body(es_ref, ht_ref, w3_ref, b3_ref, es2_ref, b3s_ref):
        e2 = es_ref[...] + jnp.maximum(ht_ref[...], 0.0)
        es2_ref[...] = e2
        b3s_ref[...] = _dot(e2, w3_ref[...]) + b3_ref[...]

    espec = pl.BlockSpec((EB, HID), lambda i: (i, 0))
    return pl.pallas_call(
        body,
        grid=(GRID_E,),
        in_specs=[espec, espec,
                  pl.BlockSpec((HID, HID), lambda i: (0, 0)),
                  pl.BlockSpec((1, HID), lambda i: (0, 0))],
        out_specs=[espec, espec],
        out_shape=[
            jax.ShapeDtypeStruct((EP, HID), F32),
            jax.ShapeDtypeStruct((EP, HID), F32),
        ],
    )(es, hats, w3, b3)


def _k_node_update(h, p, a1h, c1p, acchs, accps):
    def body(h_ref, p_ref, a1_ref, c1_ref, ah_ref, ap_ref, h2_ref, p2_ref):
        acch = jnp.concatenate([ah_ref[q, :, :] for q in range(4)], axis=1)
        accp = jnp.concatenate([ap_ref[q, :, :] for q in range(4)], axis=1)
        h2_ref[...] = h_ref[...] + jnp.maximum(a1_ref[...] + acch, 0.0)
        p2_ref[...] = p_ref[...] + jnp.tanh(c1_ref[...] + accp)

    nspec = pl.BlockSpec((NB, HID), lambda i: (i, 0))
    sspec = pl.BlockSpec((4, NB, QW), lambda i: (0, i, 0))
    return pl.pallas_call(
        body,
        grid=(GRID_N,),
        in_specs=[nspec, nspec, nspec, nspec, sspec, sspec],
        out_specs=[nspec, nspec],
        out_shape=[
            jax.ShapeDtypeStruct((NP, HID), F32),
            jax.ShapeDtypeStruct((NP, HID), F32),
        ],
    )(h, p, a1h, c1p, acchs, accps)


def _oh_g(gid_ref):
    return (gid_ref[...] ==
            lax.broadcasted_iota(jnp.int32, (1, G), 1)).astype(F32)


def _k_read1(p4, gid2d, wpo, bpo):
    def body(p4_ref, gid_ref, w_ref, b_ref, pl_ref, sums_ref, cnt_ref):
        i = pl.program_id(0)
        oh = _oh_g(gid_ref)
        plv = _dot(p4_ref[...], w_ref[...]) + b_ref[...]
        pl_ref[...] = plv

        @pl.when(i == 0)
        def _():
            sums_ref[...] = jnp.zeros_like(sums_ref)
            cnt_ref[...] = jnp.zeros_like(cnt_ref)
        sums_ref[...] += _dotr(oh, plv)
        cnt_ref[...] += _dotr(oh, jnp.ones((NB, 8), F32))

    return pl.pallas_call(
        body,
        grid=(GRID_N,),
        in_specs=[
            pl.BlockSpec((NB, HID), lambda i: (i, 0)),
            pl.BlockSpec((NB, 1), lambda i: (i, 0)),
            pl.BlockSpec((HID, PE), lambda i: (0, 0)),
            pl.BlockSpec((1, PE), lambda i: (0, 0)),
        ],
        out_specs=[
            pl.BlockSpec((NB, PE), lambda i: (i, 0)),
            pl.BlockSpec((G, PE), lambda i: (0, 0)),
            pl.BlockSpec((G, 8), lambda i: (0, 0)),
        ],
        out_shape=[
            jax.ShapeDtypeStruct((NP, PE), F32),
            jax.ShapeDtypeStruct((G, PE), F32),
            jax.ShapeDtypeStruct((G, 8), F32),
        ],
    )(p4, gid2d, wpo, bpo)


def _k_read2(pL, gid2d, sums, cnt):
    def body(pl_ref, gid_ref, sums_ref, cnt_ref, pc_ref, nsq_ref):
        i = pl.program_id(0)
        oh = _oh_g(gid_ref)
        means = sums_ref[...] / jnp.maximum(cnt_ref[:, :1], 1.0)
        pc = pl_ref[...] - _dotx(oh, means)
        pc_ref[...] = pc

        @pl.when(i == 0)
        def _():
            nsq_ref[...] = jnp.zeros_like(nsq_ref)
        nsq_ref[...] += _dotr(oh, pc * pc)

    return pl.pallas_call(
        body,
        grid=(GRID_N,),
        in_specs=[
            pl.BlockSpec((NB, PE), lambda i: (i, 0)),
            pl.BlockSpec((NB, 1), lambda i: (i, 0)),
            pl.BlockSpec((G, PE), lambda i: (0, 0)),
            pl.BlockSpec((G, 8), lambda i: (0, 0)),
        ],
        out_specs=[
            pl.BlockSpec((NB, PE), lambda i: (i, 0)),
            pl.BlockSpec((G, PE), lambda i: (0, 0)),
        ],
        out_shape=[
            jax.ShapeDtypeStruct((NP, PE), F32),
            jax.ShapeDtypeStruct((G, PE), F32),
        ],
    )(pL, gid2d, sums, cnt)


def _k_read3(pc, gid2d, nsq, h4, whp, bhp):
    def body(pc_ref, gid_ref, nsq_ref, h4_ref, w_ref, b_ref, hg_ref):
        i = pl.program_id(0)
        oh = _oh_g(gid_ref)
        rn = lax.rsqrt(jnp.maximum(nsq_ref[...], 1e-30))
        pn = pc_ref[...] * _dotx(oh, rn)
        hpv = jnp.concatenate([h4_ref[...], pn], axis=1)
        hpw = _dot(hpv, w_ref[...]) + b_ref[...]

        @pl.when(i == 0)
        def _():
            hg_ref[...] = jnp.zeros_like(hg_ref)
        hg_ref[...] += _dotr(oh, hpw)

    return pl.pallas_call(
        body,
        grid=(GRID_N,),
        in_specs=[
            pl.BlockSpec((NB, PE), lambda i: (i, 0)),
            pl.BlockSpec((NB, 1), lambda i: (i, 0)),
            pl.BlockSpec((G, PE), lambda i: (0, 0)),
            pl.BlockSpec((NB, HID), lambda i: (i, 0)),
            pl.BlockSpec((HID + PE, HID), lambda i: (0, 0)),
            pl.BlockSpec((1, HID), lambda i: (0, 0)),
        ],
        out_specs=[pl.BlockSpec((G, HID), lambda i: (0, 0))],
        out_shape=[jax.ShapeDtypeStruct((G, HID), F32)],
    )(pc, gid2d, nsq, h4, whp, bhp)


def _k_read4(hgsum, cnt, params):
    def body(hg_ref, cnt_ref, w0, b0, w1, b1, w2, b2, y_ref):
        hg = hg_ref[...] / jnp.maximum(cnt_ref[:, :1], 1.0)
        y = jnp.maximum(_dot(hg, w0[...]) + b0[...], 0.0)
        y = jnp.maximum(_dot(y, w1[...]) + b1[...], 0.0)
        y_ref[...] = _dot(y, w2[...]) + b2[...]

    return pl.pallas_call(
        body,
        out_shape=jax.ShapeDtypeStruct((G, 1), F32),
    )(hgsum, cnt,
      params['mlp0']['w'], params['mlp0']['b'][None, :],
      params['mlp1']['w'], params['mlp1']['b'][None, :],
      params['mlp2']['w'], params['mlp2']['b'][None, :])


def kernel(h, p, e, snorm_n, edge_index, graph_ids, params):
    src = edge_index[0]
    dst = edge_index[1]
    src2d = jnp.pad(src, (0, EP - E), constant_values=N).reshape(EP // 128, 128)
    dst2d = jnp.pad(dst, (0, EP - E), constant_values=N).reshape(EP // 128, 128)
    eids2d = jnp.pad(e, (0, EP - E))[:, None]
    ids2d = jnp.pad(h, (0, NP - N))[:, None]
    p_pad = jnp.pad(p, ((0, NP - N), (0, 0)))
    gid2d = jnp.pad(graph_ids, (0, NP - N), constant_values=G)[:, None]

    hc, pc = _k_embed(ids2d, p_pad, params['emb_h'],
                      params['emb_p']['w'], params['emb_p']['b'][None, :])
    lp0 = params['layers'][0]
    es, b3s = _k_edge_embed(eids2d, params['emb_e'],
                            lp0['B3']['w'], lp0['B3']['b'][None, :])

    for li in range(4):
        lp = params['layers'][li]
        a1h, c1p, b1s, b2s, a2s, c2s = _k_node_pre(hc, pc, lp)
        hatf, ssf = _sc_pass1(src2d, dst2d,
                              b1s.reshape(4 * NP, QW),
                              b2s.reshape(4 * NP, QW),
                              b3s)
        etaf, acch = _sc_pass2(src2d, dst2d, hatf, ssf,
                               a2s.reshape(4 * NP, QW))
        accp = _sc_pass3(src2d, dst2d, etaf, c2s.reshape(4 * NP, QW))
        hc, pc = _k_node_update(hc, pc, a1h, c1p,
                                acch.reshape(4, NP, QW),
                                accp.reshape(4, NP, QW))
        if li < 3:
            nlp = params['layers'][li + 1]
            es, b3s = _k_edge_update(es, hatf,
                                     nlp['B3']['w'], nlp['B3']['b'][None, :])

    pL, sums, cnt = _k_read1(pc, gid2d, params['p_out']['w'],
                             params['p_out']['b'][None, :])
    pcC, nsq = _k_read2(pL, gid2d, sums, cnt)
    hgsum, = _k_read3(pcC, gid2d, nsq, hc, params['Whp']['w'],
                      params['Whp']['b'][None, :])
    return _k_read4(hgsum, cnt, params)
